# baseline (speedup 1.0000x reference)
import functools

import numpy as np
import jax
import jax.numpy as jnp
from jax import lax
from jax.experimental import pallas as pl
from jax.experimental.pallas import tpu as pltpu
from jax.experimental.pallas import tpu_sc as plsc

D_H = 128
N_RBF = 16
N_NODES = 10000
N_EDGES = 640000
_SQRT2 = float(np.sqrt(2.0))

NC = 2
NS = 16
NW = NC * NS
NPAD = 10240
STRIP = NPAD // NS

E_PER_W = N_EDGES // NW
EC = 80
NCHUNK = E_PER_W // EC
EC0 = 160
NCHUNK0 = E_PER_W // EC0

_CENTERS = np.linspace(0.0, _SQRT2, N_RBF).astype(np.float32).reshape(1, N_RBF)
_WIDTH = _SQRT2 / (N_RBF - 1)
_NEG_INV_2W2 = np.float32(-1.0 / (2.0 * _WIDTH * _WIDTH))
_LOG_N = float(np.log(float(N_NODES)))


def _sc_mesh():
    return plsc.VectorSubcoreMesh(
        core_axis_name="c", subcore_axis_name="s", num_cores=NC, num_subcores=NS
    )


def _sc_pass0(x, y, src, dst, w):
    @functools.partial(
        pl.kernel,
        out_type=(
            jax.ShapeDtypeStruct((N_EDGES,), jnp.float32),
            jax.ShapeDtypeStruct((NC, NPAD), jnp.float32),
        ),
        mesh=_sc_mesh(),
        scratch_types=[
            pltpu.VMEM((N_NODES,), jnp.float32),
            pltpu.VMEM((N_NODES,), jnp.float32),
            pltpu.VMEM((EC0,), jnp.int32),
            pltpu.VMEM((EC0,), jnp.int32),
            pltpu.VMEM((EC0,), jnp.float32),
            pltpu.VMEM((EC0,), jnp.float32),
            pltpu.VMEM((STRIP,), jnp.float32),
            pltpu.VMEM_SHARED((NPAD,), jnp.float32),
        ],
        compiler_params=pltpu.CompilerParams(needs_layout_passes=False),
    )
    def k(x_h, y_h, src_h, dst_h, w_h, r2_h, ws_h,
          x_v, y_v, src_v, dst_v, w_v, r2_v, z_v, ws_sh):
        cid = lax.axis_index("c")
        tid = lax.axis_index("s")
        wid = cid * NS + tid

        pltpu.sync_copy(x_h, x_v)
        pltpu.sync_copy(y_h, y_v)

        zero16 = jnp.zeros((16,), jnp.float32)

        def zz(i, carry):
            z_v[pl.ds(i * 16, 16)] = zero16
            return carry

        lax.fori_loop(0, STRIP // 16, zz, 0)
        pltpu.sync_copy(z_v, ws_sh.at[pl.ds(tid * STRIP, STRIP)])
        plsc.subcore_barrier()

        def chunk(g, carry):
            base = wid * E_PER_W + g * EC0
            pltpu.sync_copy(src_h.at[pl.ds(base, EC0)], src_v)
            pltpu.sync_copy(dst_h.at[pl.ds(base, EC0)], dst_v)
            pltpu.sync_copy(w_h.at[pl.ds(base, EC0)], w_v)

            def grp(i, c2):
                s16 = src_v[pl.ds(i * 16, 16)]
                d16 = dst_v[pl.ds(i * 16, 16)]
                xs = plsc.load_gather(x_v, [s16])
                xd = plsc.load_gather(x_v, [d16])
                ys = plsc.load_gather(y_v, [s16])
                yd = plsc.load_gather(y_v, [d16])
                dx = xs - xd
                dy = ys - yd
                r2_v[pl.ds(i * 16, 16)] = dx * dx + dy * dy
                return c2

            lax.fori_loop(0, EC0 // 16, grp, 0)
            pltpu.sync_copy(r2_v, r2_h.at[pl.ds(base, EC0)])
            pltpu.sync_copy(w_v, ws_sh.at[dst_v], add=True)
            return carry

        lax.fori_loop(0, NCHUNK0, chunk, 0)
        plsc.subcore_barrier()
        pltpu.sync_copy(ws_sh.at[pl.ds(tid * STRIP, STRIP)],
                        ws_h.at[cid, pl.ds(tid * STRIP, STRIP)])

    return k(x, y, src, dst, w)


def _sc_layer(a, b, c, sdw):
    @functools.partial(
        pl.kernel,
        out_type=jax.ShapeDtypeStruct((NC, NPAD, D_H), jnp.float32),
        mesh=_sc_mesh(),
        scratch_types=[
            pltpu.VMEM((2, 3, EC), jnp.int32),
            pltpu.VMEM((2, EC, D_H), jnp.float32),
            pltpu.VMEM((2, EC, D_H), jnp.float32),
            pltpu.VMEM_SHARED((NPAD, D_H), jnp.float32),
            pltpu.SemaphoreType.DMA,
            pltpu.SemaphoreType.DMA,
            pltpu.SemaphoreType.DMA,
            pltpu.SemaphoreType.DMA,
            pltpu.SemaphoreType.DMA,
            pltpu.SemaphoreType.DMA,
            pltpu.SemaphoreType.DMA,
            pltpu.SemaphoreType.DMA,
        ],
        compiler_params=pltpu.CompilerParams(needs_layout_passes=False),
    )
    def k(a_h, b_h, c_h, sdw_h, s_out,
          idx_v, m_v, b_v, s_sh,
          si0, si1, sc0, sc1, sg0, sg1, ss0, ss1):
        cid = lax.axis_index("c")
        tid = lax.axis_index("s")
        wid = cid * NS + tid
        sem_i = (si0, si1)
        sem_c = (sc0, sc1)
        sem_g = (sg0, sg1)
        sem_s = (ss0, ss1)

        zero16 = jnp.zeros((16,), jnp.float32)

        def zz(i, carry):
            for j in range(D_H // 16):
                m_v[0, i, pl.ds(j * 16, 16)] = zero16
            return carry

        lax.fori_loop(0, EC, zz, 0)
        for kk in range(STRIP // EC):
            pltpu.sync_copy(m_v.at[0],
                            s_sh.at[pl.ds(tid * STRIP + kk * EC, EC)])
        plsc.subcore_barrier()

        def issue_idx_c(g, p):
            base = wid * E_PER_W + g * EC
            pltpu.async_copy(sdw_h.at[wid * NCHUNK + g], idx_v.at[p],
                             sem_i[p])
            pltpu.async_copy(c_h.at[pl.ds(base, EC)], m_v.at[p], sem_c[p])

        def wait_idx_c(p):
            pltpu.make_async_copy(sdw_h.at[0], idx_v.at[p], sem_i[p]).wait()
            pltpu.make_async_copy(c_h.at[pl.ds(0, EC)], m_v.at[p],
                                  sem_c[p]).wait()

        def issue_ga(p):
            pltpu.async_copy(a_h.at[idx_v.at[p, 0]], m_v.at[p], sem_g[p],
                             add=True)
            pltpu.async_copy(b_h.at[idx_v.at[p, 1]], b_v.at[p], sem_g[p])

        def wait_ga(p):
            pltpu.make_async_copy(a_h.at[idx_v.at[p, 0]], m_v.at[p],
                                  sem_g[p]).wait()
            pltpu.make_async_copy(b_h.at[idx_v.at[p, 1]], b_v.at[p],
                                  sem_g[p]).wait()

        def compute(p):
            def grp(g16, carry):
                pos = (jnp.full((16,), g16 * 16, jnp.int32)
                       + lax.iota(jnp.int32, 16))
                w16 = plsc.bitcast(
                    plsc.load_gather(idx_v.at[p, 2], [pos]), jnp.float32)
                for l in range(16):
                    e = g16 * 16 + l
                    wv = jnp.take_along_axis(
                        w16, jnp.full((16,), l, jnp.int32), axis=0)
                    for j in range(D_H // 16):
                        sl = pl.ds(j * 16, 16)
                        m_v[p, e, sl] = jnp.maximum(
                            (m_v[p, e, sl] + b_v[p, e, sl]) * wv, 0.0)
                return carry

            lax.fori_loop(0, EC // 16, grp, 0)

        def issue_scatter(p):
            pltpu.async_copy(m_v.at[p], s_sh.at[idx_v.at[p, 1]], sem_s[p],
                             add=True)

        def wait_scatter(p):
            pltpu.make_async_copy(m_v.at[p], s_sh.at[idx_v.at[p, 1]],
                                  sem_s[p]).wait()

        issue_idx_c(0, 0)
        issue_idx_c(1, 1)
        wait_idx_c(0)
        issue_ga(0)

        last = NCHUNK // 2 - 1

        def pair(g2, carry):
            ga = g2 * 2

            wait_idx_c(1)
            issue_ga(1)
            wait_ga(0)
            compute(0)
            issue_scatter(0)
            wait_ga(1)
            compute(1)
            issue_scatter(1)
            wait_scatter(0)

            @pl.when(g2 < last)
            def _():
                issue_idx_c(ga + 2, 0)
                wait_idx_c(0)
                issue_ga(0)
            wait_scatter(1)

            @pl.when(g2 < last)
            def _():
                issue_idx_c(ga + 3, 1)
            return carry

        lax.fori_loop(0, NCHUNK // 2, pair, 0)
        plsc.subcore_barrier()
        for kk in range(STRIP // 128):
            base = tid * STRIP + kk * 128
            pltpu.sync_copy(s_sh.at[pl.ds(base, 128)],
                            s_out.at[cid, pl.ds(base, 128)])

    return k(a, b, c, sdw)


def _tc_rbf(r2, w1r_all, b1_all):
    EB = 2000

    def body(r2_ref, wr_ref, br_ref, c1_ref, c2_ref, c3_ref):
        r = jnp.sqrt(r2_ref[...] + 1e-8)
        centers = (lax.broadcasted_iota(jnp.int32, (1, N_RBF), 1)
                   .astype(jnp.float32) * np.float32(_SQRT2 / (N_RBF - 1)))
        d = r - centers
        rbf = jnp.exp(d * d * _NEG_INV_2W2)
        cc = jnp.dot(rbf, wr_ref[...],
                     preferred_element_type=jnp.float32) + br_ref[...]
        c1_ref[...] = cc[:, :D_H]
        c2_ref[...] = cc[:, D_H:2 * D_H]
        c3_ref[...] = cc[:, 2 * D_H:]

    return pl.pallas_call(
        body,
        grid=(N_EDGES // EB,),
        in_specs=[
            pl.BlockSpec((EB, 1), lambda i: (i, 0)),
            pl.BlockSpec((N_RBF, 3 * D_H), lambda i: (0, 0)),
            pl.BlockSpec((1, 3 * D_H), lambda i: (0, 0)),
        ],
        out_specs=[pl.BlockSpec((EB, D_H), lambda i: (i, 0))] * 3,
        out_shape=[jax.ShapeDtypeStruct((N_EDGES, D_H), jnp.float32)] * 3,
    )(r2, w1r_all, b1_all)


def _tc_node_in(xyr01, ws, w0, b0, w1, b1, wab):
    def body(xyr_ref, ws_ref, w0_ref, b0_ref, w1_ref, b1_ref, wab_ref,
             h_ref, a_ref, b_ref):
        ws_v = ws_ref[...]
        deg = ws_v / jnp.maximum(jnp.mean(ws_v), 1.0)
        x0 = jnp.concatenate([xyr_ref[...], deg], axis=1)
        t = jnp.maximum(
            jnp.dot(x0, w0_ref[...], preferred_element_type=jnp.float32)
            + b0_ref[...], 0.0)
        h = jnp.dot(t, w1_ref[...],
                    preferred_element_type=jnp.float32) + b1_ref[...]
        h_ref[...] = h
        ab = jnp.dot(h, wab_ref[...], preferred_element_type=jnp.float32)
        a_ref[...] = ab[:, :D_H]
        b_ref[...] = ab[:, D_H:]

    return pl.pallas_call(
        body,
        out_shape=[jax.ShapeDtypeStruct((N_NODES, D_H), jnp.float32)] * 3,
    )(xyr01, ws, w0, b0, w1, b1, wab)


def _tc_post(h, s2, ws, w2, b2, v1, c1, v2, c2, lng, lnb, wab_next):
    has_next = wab_next is not None

    def body(*refs):
        (h_ref, s_ref, ws_ref, w2_ref, b2_ref, v1_ref, c1_ref, v2_ref,
         c2_ref, g_ref, bb_ref) = refs[:11]
        if has_next:
            wab_ref, hn_ref, a_ref, b_ref = refs[11:]
        else:
            hn_ref = refs[11]
        h_v = h_ref[...]
        s = s_ref[0, :N_NODES, :] + s_ref[1, :N_NODES, :]
        agg = (jnp.dot(s, w2_ref[...], preferred_element_type=jnp.float32)
               + ws_ref[...] * b2_ref[...])
        cat = jnp.concatenate([h_v, agg], axis=1)
        t = jnp.maximum(
            jnp.dot(cat, v1_ref[...], preferred_element_type=jnp.float32)
            + c1_ref[...], 0.0)
        u = jnp.dot(t, v2_ref[...],
                    preferred_element_type=jnp.float32) + c2_ref[...]
        y = h_v + u
        mu = jnp.mean(y, axis=1, keepdims=True)
        var = jnp.mean((y - mu) ** 2, axis=1, keepdims=True)
        hn = (y - mu) / jnp.sqrt(var + 1e-5) * g_ref[...] + bb_ref[...]
        hn_ref[...] = hn
        if has_next:
            ab = jnp.dot(hn, wab_ref[...], preferred_element_type=jnp.float32)
            a_ref[...] = ab[:, :D_H]
            b_ref[...] = ab[:, D_H:]

    n_out = 3 if has_next else 1
    args = [h, s2, ws, w2, b2, v1, c1, v2, c2, lng, lnb]
    if has_next:
        args.append(wab_next)
    return pl.pallas_call(
        body,
        out_shape=[jax.ShapeDtypeStruct((N_NODES, D_H), jnp.float32)] * n_out,
    )(*args)


def _tc_head(h, xyr01, ws, wh0, bh0, wh1, bh1):
    def body(h_ref, xyr_ref, ws_ref, w0_ref, b0_ref, w1_ref, b1_ref, o_ref):
        h_v = h_ref[...]
        pm = jnp.mean(h_v, axis=0, keepdims=True)
        px = jnp.max(h_v, axis=0, keepdims=True)
        xyr = xyr_ref[...]
        one = jnp.ones((1, 1), jnp.float32)
        em = jnp.sum(ws_ref[...]) * (0.5 / float(N_NODES))
        col2 = xyr[:, 2:3]
        mu2 = jnp.mean(col2)
        sd2 = jnp.sqrt(jnp.mean((col2 - mu2) ** 2))
        xy = xyr[:, :2]
        muxy = jnp.mean(xy)
        sdxy = jnp.sqrt(jnp.mean((xy - muxy) ** 2))
        stats = jnp.concatenate(
            [one * _LOG_N, one * em, one * mu2, one * sd2, one * sdxy], axis=1)
        z = jnp.concatenate([pm, px, stats], axis=1)
        t = jnp.maximum(
            jnp.dot(z, w0_ref[...], preferred_element_type=jnp.float32)
            + b0_ref[...], 0.0)
        o_ref[...] = jnp.dot(t, w1_ref[...],
                             preferred_element_type=jnp.float32) + b1_ref[...]

    return pl.pallas_call(
        body,
        out_shape=jax.ShapeDtypeStruct((1, 1), jnp.float32),
    )(h, xyr01, ws, wh0, bh0, wh1, bh1)


def kernel(xyr01, edge_index, edge_weight, params):
    x = xyr01[:, 0]
    y = xyr01[:, 1]
    src = edge_index[0]
    dst = edge_index[1]
    w = edge_weight

    node_in = params['node_in']
    w0 = node_in[0]['W']
    b0 = node_in[0]['b'].reshape(1, D_H)
    w1 = node_in[1]['W']
    b1n = node_in[1]['b'].reshape(1, D_H)

    layers = params['layers']
    wab, w1r, b1m, w2, b2, v1, c1, v2, c2, lng, lnb = ([] for _ in range(11))
    for lp in layers:
        W1 = lp['phi_m'][0]['W']
        wab.append(jnp.concatenate([W1[:D_H], W1[D_H:2 * D_H]], axis=1))
        w1r.append(W1[2 * D_H:])
        b1m.append(lp['phi_m'][0]['b'])
        w2.append(lp['phi_m'][1]['W'])
        b2.append(lp['phi_m'][1]['b'].reshape(1, D_H))
        v1.append(lp['phi_h'][0]['W'])
        c1.append(lp['phi_h'][0]['b'].reshape(1, D_H))
        v2.append(lp['phi_h'][1]['W'])
        c2.append(lp['phi_h'][1]['b'].reshape(1, D_H))
        lng.append(lp['ln_scale'].reshape(1, D_H))
        lnb.append(lp['ln_bias'].reshape(1, D_H))
    w1r_all = jnp.concatenate(w1r, axis=1)
    b1_all = jnp.concatenate(b1m, axis=0).reshape(1, 3 * D_H)

    head = params['head']
    wh0 = head[0]['W']
    bh0 = head[0]['b'].reshape(1, D_H)
    wh1 = head[1]['W']
    bh1 = head[1]['b'].reshape(1, 1)

    sdw = jnp.stack(
        [src.reshape(-1, EC), dst.reshape(-1, EC),
         lax.bitcast_convert_type(w, jnp.int32).reshape(-1, EC)], axis=1)

    r2, ws2 = _sc_pass0(x, y, src, dst, w)
    ws = (ws2[0] + ws2[1])[:N_NODES].reshape(N_NODES, 1)

    c_all = _tc_rbf(r2.reshape(N_EDGES, 1), w1r_all, b1_all)

    h, a_n, b_n = _tc_node_in(xyr01, ws, w0, b0, w1, b1n, wab[0])

    for l in range(len(layers)):
        s2 = _sc_layer(a_n, b_n, c_all[l], sdw)
        nxt = wab[l + 1] if l + 1 < len(layers) else None
        outs = _tc_post(h, s2, ws, w2[l], b2[l], v1[l], c1[l], v2[l], c2[l],
                        lng[l], lnb[l], nxt)
        if nxt is not None:
            h, a_n, b_n = outs
        else:
            h, = outs

    out = _tc_head(h, xyr01, ws, wh0, bh0, wh1, bh1)
    return out.reshape(())


# speedup vs baseline: 1.1620x; 1.1620x over previous
import functools

import numpy as np
import jax
import jax.numpy as jnp
from jax import lax
from jax.experimental import pallas as pl
from jax.experimental.pallas import tpu as pltpu
from jax.experimental.pallas import tpu_sc as plsc

D_H = 128
N_RBF = 16
N_NODES = 10000
N_EDGES = 640000
_SQRT2 = float(np.sqrt(2.0))

NC = 2
NS = 16
NW = NC * NS
NPAD = 10240
STRIP = NPAD // NS

E_PER_W = N_EDGES // NW
EC = 80
NCHUNK = E_PER_W // EC
EC0 = 160
NCHUNK0 = E_PER_W // EC0

_CENTERS = np.linspace(0.0, _SQRT2, N_RBF).astype(np.float32).reshape(1, N_RBF)
_WIDTH = _SQRT2 / (N_RBF - 1)
_NEG_INV_2W2 = np.float32(-1.0 / (2.0 * _WIDTH * _WIDTH))
_LOG_N = float(np.log(float(N_NODES)))


def _sc_mesh():
    return plsc.VectorSubcoreMesh(
        core_axis_name="c", subcore_axis_name="s", num_cores=NC, num_subcores=NS
    )


def _sc_pass0(x, y, src, dst, w):
    @functools.partial(
        pl.kernel,
        out_type=(
            jax.ShapeDtypeStruct((N_EDGES,), jnp.float32),
            jax.ShapeDtypeStruct((NC, NPAD), jnp.float32),
        ),
        mesh=_sc_mesh(),
        scratch_types=[
            pltpu.VMEM((N_NODES,), jnp.float32),
            pltpu.VMEM((N_NODES,), jnp.float32),
            pltpu.VMEM((EC0,), jnp.int32),
            pltpu.VMEM((EC0,), jnp.int32),
            pltpu.VMEM((EC0,), jnp.float32),
            pltpu.VMEM((EC0,), jnp.float32),
            pltpu.VMEM((STRIP,), jnp.float32),
            pltpu.VMEM_SHARED((NPAD,), jnp.float32),
        ],
        compiler_params=pltpu.CompilerParams(needs_layout_passes=False),
    )
    def k(x_h, y_h, src_h, dst_h, w_h, r2_h, ws_h,
          x_v, y_v, src_v, dst_v, w_v, r2_v, z_v, ws_sh):
        cid = lax.axis_index("c")
        tid = lax.axis_index("s")
        wid = cid * NS + tid

        pltpu.sync_copy(x_h, x_v)
        pltpu.sync_copy(y_h, y_v)

        zero16 = jnp.zeros((16,), jnp.float32)

        def zz(i, carry):
            z_v[pl.ds(i * 16, 16)] = zero16
            return carry

        lax.fori_loop(0, STRIP // 16, zz, 0)
        pltpu.sync_copy(z_v, ws_sh.at[pl.ds(tid * STRIP, STRIP)])
        plsc.subcore_barrier()

        def chunk(g, carry):
            base = wid * E_PER_W + g * EC0
            pltpu.sync_copy(src_h.at[pl.ds(base, EC0)], src_v)
            pltpu.sync_copy(dst_h.at[pl.ds(base, EC0)], dst_v)
            pltpu.sync_copy(w_h.at[pl.ds(base, EC0)], w_v)

            def grp(i, c2):
                s16 = src_v[pl.ds(i * 16, 16)]
                d16 = dst_v[pl.ds(i * 16, 16)]
                xs = plsc.load_gather(x_v, [s16])
                xd = plsc.load_gather(x_v, [d16])
                ys = plsc.load_gather(y_v, [s16])
                yd = plsc.load_gather(y_v, [d16])
                dx = xs - xd
                dy = ys - yd
                r2_v[pl.ds(i * 16, 16)] = dx * dx + dy * dy
                return c2

            lax.fori_loop(0, EC0 // 16, grp, 0)
            pltpu.sync_copy(r2_v, r2_h.at[pl.ds(base, EC0)])
            pltpu.sync_copy(w_v, ws_sh.at[dst_v], add=True)
            return carry

        lax.fori_loop(0, NCHUNK0, chunk, 0)
        plsc.subcore_barrier()
        pltpu.sync_copy(ws_sh.at[pl.ds(tid * STRIP, STRIP)],
                        ws_h.at[cid, pl.ds(tid * STRIP, STRIP)])

    return k(x, y, src, dst, w)


def _sc_layer(a, b, c, sdw):
    @functools.partial(
        pl.kernel,
        out_type=jax.ShapeDtypeStruct((NC, NPAD, D_H), jnp.float32),
        mesh=_sc_mesh(),
        scratch_types=(
            [pltpu.VMEM((4, 3, EC), jnp.int32),
             pltpu.VMEM((4, EC, D_H), jnp.float32),
             pltpu.VMEM_SHARED((NPAD, D_H), jnp.float32)]
            + [pltpu.SemaphoreType.DMA] * 16
        ),
        compiler_params=pltpu.CompilerParams(needs_layout_passes=False),
    )
    def k(a_h, b_h, c_h, sdw_h, s_out,
          idx_v, m_v, s_sh, *sems):
        cid = lax.axis_index("c")
        tid = lax.axis_index("s")
        wid = cid * NS + tid
        sem_ic = sems[0:4]
        sem_a = sems[4:8]
        sem_b = sems[8:12]
        sem_s = sems[12:16]

        zero16 = jnp.zeros((16,), jnp.float32)

        def zz(i, carry):
            for j in range(D_H // 16):
                m_v[0, i, pl.ds(j * 16, 16)] = zero16
            return carry

        lax.fori_loop(0, EC, zz, 0)
        for kk in range(STRIP // EC):
            pltpu.sync_copy(m_v.at[0],
                            s_sh.at[pl.ds(tid * STRIP + kk * EC, EC)])
        plsc.subcore_barrier()

        def issue_idx_c(g, p):
            base = wid * E_PER_W + g * EC
            pltpu.async_copy(sdw_h.at[wid * NCHUNK + g], idx_v.at[p],
                             sem_ic[p])
            pltpu.async_copy(c_h.at[pl.ds(base, EC)], m_v.at[p], sem_ic[p])

        def wait_idx_c(p):
            pltpu.make_async_copy(sdw_h.at[0], idx_v.at[p], sem_ic[p]).wait()
            pltpu.make_async_copy(c_h.at[pl.ds(0, EC)], m_v.at[p],
                                  sem_ic[p]).wait()

        def issue_a(p):
            pltpu.async_copy(a_h.at[idx_v.at[p, 0]], m_v.at[p], sem_a[p],
                             add=True)

        def wait_a(p):
            pltpu.make_async_copy(a_h.at[idx_v.at[p, 0]], m_v.at[p],
                                  sem_a[p]).wait()

        def issue_b(p):
            pltpu.async_copy(b_h.at[idx_v.at[p, 1]], m_v.at[p], sem_b[p],
                             add=True)

        def wait_b(p):
            pltpu.make_async_copy(b_h.at[idx_v.at[p, 1]], m_v.at[p],
                                  sem_b[p]).wait()

        def compute(p):
            def grp(g16, carry):
                pos = (jnp.full((16,), g16 * 16, jnp.int32)
                       + lax.iota(jnp.int32, 16))
                w16 = plsc.bitcast(
                    plsc.load_gather(idx_v.at[p, 2], [pos]), jnp.float32)
                for l in range(16):
                    e = g16 * 16 + l
                    wv = jnp.take_along_axis(
                        w16, jnp.full((16,), l, jnp.int32), axis=0)
                    for j in range(D_H // 16):
                        sl = pl.ds(j * 16, 16)
                        m_v[p, e, sl] = jnp.maximum(m_v[p, e, sl] * wv, 0.0)
                return carry

            lax.fori_loop(0, EC // 16, grp, 0)

        def issue_scatter(p):
            pltpu.async_copy(m_v.at[p], s_sh.at[idx_v.at[p, 1]], sem_s[p],
                             add=True)

        def wait_scatter(p):
            pltpu.make_async_copy(m_v.at[p], s_sh.at[idx_v.at[p, 1]],
                                  sem_s[p]).wait()

        def chunk(g, p, first=False, has1=True, has2=True, has3=True):
            wait_b(p)
            compute(p)
            issue_scatter(p)
            if has1:
                wait_a((p + 1) % 4)
                issue_b((p + 1) % 4)
            if has2:
                wait_idx_c((p + 2) % 4)
                issue_a((p + 2) % 4)
            if not first:
                wait_scatter((p + 3) % 4)
                if has3:
                    issue_idx_c(g + 3, (p + 3) % 4)
            elif has3:
                issue_idx_c(g + 3, (p + 3) % 4)

        issue_idx_c(0, 0)
        issue_idx_c(1, 1)
        issue_idx_c(2, 2)
        wait_idx_c(0)
        issue_a(0)
        wait_a(0)
        issue_b(0)
        wait_idx_c(1)
        issue_a(1)

        chunk(0, 0, first=True)

        def blk(i, carry):
            g = 1 + i * 4
            chunk(g, 1)
            chunk(g + 1, 2)
            chunk(g + 2, 3)
            chunk(g + 3, 0)
            return carry

        lax.fori_loop(0, (NCHUNK - 6) // 4, blk, 0)
        for g in range(NCHUNK - 5, NCHUNK):
            chunk(g, g % 4, has1=g + 1 < NCHUNK, has2=g + 2 < NCHUNK,
                  has3=g + 3 < NCHUNK)
        wait_scatter((NCHUNK - 1) % 4)
        plsc.subcore_barrier()
        for kk in range(STRIP // 128):
            base = tid * STRIP + kk * 128
            pltpu.sync_copy(s_sh.at[pl.ds(base, 128)],
                            s_out.at[cid, pl.ds(base, 128)])

    return k(a, b, c, sdw)


def _tc_rbf(r2, w1r_all, b1_all):
    EB = 2000

    def body(r2_ref, wr_ref, br_ref, c1_ref, c2_ref, c3_ref):
        r = jnp.sqrt(r2_ref[...] + 1e-8)
        centers = (lax.broadcasted_iota(jnp.int32, (1, N_RBF), 1)
                   .astype(jnp.float32) * np.float32(_SQRT2 / (N_RBF - 1)))
        d = r - centers
        rbf = jnp.exp(d * d * _NEG_INV_2W2)
        cc = jnp.dot(rbf, wr_ref[...],
                     preferred_element_type=jnp.float32) + br_ref[...]
        c1_ref[...] = cc[:, :D_H]
        c2_ref[...] = cc[:, D_H:2 * D_H]
        c3_ref[...] = cc[:, 2 * D_H:]

    return pl.pallas_call(
        body,
        grid=(N_EDGES // EB,),
        in_specs=[
            pl.BlockSpec((EB, 1), lambda i: (i, 0)),
            pl.BlockSpec((N_RBF, 3 * D_H), lambda i: (0, 0)),
            pl.BlockSpec((1, 3 * D_H), lambda i: (0, 0)),
        ],
        out_specs=[pl.BlockSpec((EB, D_H), lambda i: (i, 0))] * 3,
        out_shape=[jax.ShapeDtypeStruct((N_EDGES, D_H), jnp.float32)] * 3,
    )(r2, w1r_all, b1_all)


def _tc_node_in(xyr01, ws, w0, b0, w1, b1, wab):
    def body(xyr_ref, ws_ref, w0_ref, b0_ref, w1_ref, b1_ref, wab_ref,
             h_ref, a_ref, b_ref):
        ws_v = ws_ref[...]
        deg = ws_v / jnp.maximum(jnp.mean(ws_v), 1.0)
        x0 = jnp.concatenate([xyr_ref[...], deg], axis=1)
        t = jnp.maximum(
            jnp.dot(x0, w0_ref[...], preferred_element_type=jnp.float32)
            + b0_ref[...], 0.0)
        h = jnp.dot(t, w1_ref[...],
                    preferred_element_type=jnp.float32) + b1_ref[...]
        h_ref[...] = h
        ab = jnp.dot(h, wab_ref[...], preferred_element_type=jnp.float32)
        a_ref[...] = ab[:, :D_H]
        b_ref[...] = ab[:, D_H:]

    return pl.pallas_call(
        body,
        out_shape=[jax.ShapeDtypeStruct((N_NODES, D_H), jnp.float32)] * 3,
    )(xyr01, ws, w0, b0, w1, b1, wab)


def _tc_post(h, s2, ws, w2, b2, v1, c1, v2, c2, lng, lnb, wab_next):
    has_next = wab_next is not None

    def body(*refs):
        (h_ref, s_ref, ws_ref, w2_ref, b2_ref, v1_ref, c1_ref, v2_ref,
         c2_ref, g_ref, bb_ref) = refs[:11]
        if has_next:
            wab_ref, hn_ref, a_ref, b_ref = refs[11:]
        else:
            hn_ref = refs[11]
        h_v = h_ref[...]
        s = s_ref[0, :N_NODES, :] + s_ref[1, :N_NODES, :]
        agg = (jnp.dot(s, w2_ref[...], preferred_element_type=jnp.float32)
               + ws_ref[...] * b2_ref[...])
        cat = jnp.concatenate([h_v, agg], axis=1)
        t = jnp.maximum(
            jnp.dot(cat, v1_ref[...], preferred_element_type=jnp.float32)
            + c1_ref[...], 0.0)
        u = jnp.dot(t, v2_ref[...],
                    preferred_element_type=jnp.float32) + c2_ref[...]
        y = h_v + u
        mu = jnp.mean(y, axis=1, keepdims=True)
        var = jnp.mean((y - mu) ** 2, axis=1, keepdims=True)
        hn = (y - mu) / jnp.sqrt(var + 1e-5) * g_ref[...] + bb_ref[...]
        hn_ref[...] = hn
        if has_next:
            ab = jnp.dot(hn, wab_ref[...], preferred_element_type=jnp.float32)
            a_ref[...] = ab[:, :D_H]
            b_ref[...] = ab[:, D_H:]

    n_out = 3 if has_next else 1
    args = [h, s2, ws, w2, b2, v1, c1, v2, c2, lng, lnb]
    if has_next:
        args.append(wab_next)
    return pl.pallas_call(
        body,
        out_shape=[jax.ShapeDtypeStruct((N_NODES, D_H), jnp.float32)] * n_out,
    )(*args)


def _tc_head(h, xyr01, ws, wh0, bh0, wh1, bh1):
    def body(h_ref, xyr_ref, ws_ref, w0_ref, b0_ref, w1_ref, b1_ref, o_ref):
        h_v = h_ref[...]
        pm = jnp.mean(h_v, axis=0, keepdims=True)
        px = jnp.max(h_v, axis=0, keepdims=True)
        xyr = xyr_ref[...]
        one = jnp.ones((1, 1), jnp.float32)
        em = jnp.sum(ws_ref[...]) * (0.5 / float(N_NODES))
        col2 = xyr[:, 2:3]
        mu2 = jnp.mean(col2)
        sd2 = jnp.sqrt(jnp.mean((col2 - mu2) ** 2))
        xy = xyr[:, :2]
        muxy = jnp.mean(xy)
        sdxy = jnp.sqrt(jnp.mean((xy - muxy) ** 2))
        stats = jnp.concatenate(
            [one * _LOG_N, one * em, one * mu2, one * sd2, one * sdxy], axis=1)
        z = jnp.concatenate([pm, px, stats], axis=1)
        t = jnp.maximum(
            jnp.dot(z, w0_ref[...], preferred_element_type=jnp.float32)
            + b0_ref[...], 0.0)
        o_ref[...] = jnp.dot(t, w1_ref[...],
                             preferred_element_type=jnp.float32) + b1_ref[...]

    return pl.pallas_call(
        body,
        out_shape=jax.ShapeDtypeStruct((1, 1), jnp.float32),
    )(h, xyr01, ws, wh0, bh0, wh1, bh1)


def kernel(xyr01, edge_index, edge_weight, params):
    x = xyr01[:, 0]
    y = xyr01[:, 1]
    src = edge_index[0]
    dst = edge_index[1]
    w = edge_weight

    node_in = params['node_in']
    w0 = node_in[0]['W']
    b0 = node_in[0]['b'].reshape(1, D_H)
    w1 = node_in[1]['W']
    b1n = node_in[1]['b'].reshape(1, D_H)

    layers = params['layers']
    wab, w1r, b1m, w2, b2, v1, c1, v2, c2, lng, lnb = ([] for _ in range(11))
    for lp in layers:
        W1 = lp['phi_m'][0]['W']
        wab.append(jnp.concatenate([W1[:D_H], W1[D_H:2 * D_H]], axis=1))
        w1r.append(W1[2 * D_H:])
        b1m.append(lp['phi_m'][0]['b'])
        w2.append(lp['phi_m'][1]['W'])
        b2.append(lp['phi_m'][1]['b'].reshape(1, D_H))
        v1.append(lp['phi_h'][0]['W'])
        c1.append(lp['phi_h'][0]['b'].reshape(1, D_H))
        v2.append(lp['phi_h'][1]['W'])
        c2.append(lp['phi_h'][1]['b'].reshape(1, D_H))
        lng.append(lp['ln_scale'].reshape(1, D_H))
        lnb.append(lp['ln_bias'].reshape(1, D_H))
    w1r_all = jnp.concatenate(w1r, axis=1)
    b1_all = jnp.concatenate(b1m, axis=0).reshape(1, 3 * D_H)

    head = params['head']
    wh0 = head[0]['W']
    bh0 = head[0]['b'].reshape(1, D_H)
    wh1 = head[1]['W']
    bh1 = head[1]['b'].reshape(1, 1)

    sdw = jnp.stack(
        [src.reshape(-1, EC), dst.reshape(-1, EC),
         lax.bitcast_convert_type(w, jnp.int32).reshape(-1, EC)], axis=1)

    r2, ws2 = _sc_pass0(x, y, src, dst, w)
    ws = (ws2[0] + ws2[1])[:N_NODES].reshape(N_NODES, 1)

    c_all = _tc_rbf(r2.reshape(N_EDGES, 1), w1r_all, b1_all)

    h, a_n, b_n = _tc_node_in(xyr01, ws, w0, b0, w1, b1n, wab[0])

    for l in range(len(layers)):
        s2 = _sc_layer(a_n, b_n, c_all[l], sdw)
        nxt = wab[l + 1] if l + 1 < len(layers) else None
        outs = _tc_post(h, s2, ws, w2[l], b2[l], v1[l], c1[l], v2[l], c2[l],
                        lng[l], lnb[l], nxt)
        if nxt is not None:
            h, a_n, b_n = outs
        else:
            h, = outs

    out = _tc_head(h, xyr01, ws, wh0, bh0, wh1, bh1)
    return out.reshape(())


# speedup vs baseline: 1.1717x; 1.0083x over previous
import functools

import numpy as np
import jax
import jax.numpy as jnp
from jax import lax
from jax.experimental import pallas as pl
from jax.experimental.pallas import tpu as pltpu
from jax.experimental.pallas import tpu_sc as plsc

D_H = 128
N_RBF = 16
N_NODES = 10000
N_EDGES = 640000
_SQRT2 = float(np.sqrt(2.0))

NC = 2
NS = 16
NW = NC * NS
NPAD = 10240
STRIP = NPAD // NS

E_PER_W = N_EDGES // NW
EC = 80
NCHUNK = E_PER_W // EC
EC0 = 160
NCHUNK0 = E_PER_W // EC0

_CENTERS = np.linspace(0.0, _SQRT2, N_RBF).astype(np.float32).reshape(1, N_RBF)
_WIDTH = _SQRT2 / (N_RBF - 1)
_NEG_INV_2W2 = np.float32(-1.0 / (2.0 * _WIDTH * _WIDTH))
_LOG_N = float(np.log(float(N_NODES)))


def _sc_mesh():
    return plsc.VectorSubcoreMesh(
        core_axis_name="c", subcore_axis_name="s", num_cores=NC, num_subcores=NS
    )


def _sc_pass0(x, y, src, dst, w):
    @functools.partial(
        pl.kernel,
        out_type=(
            jax.ShapeDtypeStruct((N_EDGES,), jnp.float32),
            jax.ShapeDtypeStruct((NC, NPAD), jnp.float32),
        ),
        mesh=_sc_mesh(),
        scratch_types=[
            pltpu.VMEM((N_NODES,), jnp.float32),
            pltpu.VMEM((N_NODES,), jnp.float32),
            pltpu.VMEM((EC0,), jnp.int32),
            pltpu.VMEM((EC0,), jnp.int32),
            pltpu.VMEM((EC0,), jnp.float32),
            pltpu.VMEM((EC0,), jnp.float32),
            pltpu.VMEM((STRIP,), jnp.float32),
            pltpu.VMEM_SHARED((NPAD,), jnp.float32),
        ],
        compiler_params=pltpu.CompilerParams(needs_layout_passes=False),
    )
    def k(x_h, y_h, src_h, dst_h, w_h, r2_h, ws_h,
          x_v, y_v, src_v, dst_v, w_v, r2_v, z_v, ws_sh):
        cid = lax.axis_index("c")
        tid = lax.axis_index("s")
        wid = cid * NS + tid

        pltpu.sync_copy(x_h, x_v)
        pltpu.sync_copy(y_h, y_v)

        zero16 = jnp.zeros((16,), jnp.float32)

        def zz(i, carry):
            z_v[pl.ds(i * 16, 16)] = zero16
            return carry

        lax.fori_loop(0, STRIP // 16, zz, 0)
        pltpu.sync_copy(z_v, ws_sh.at[pl.ds(tid * STRIP, STRIP)])
        plsc.subcore_barrier()

        def chunk(g, carry):
            base = wid * E_PER_W + g * EC0
            pltpu.sync_copy(src_h.at[pl.ds(base, EC0)], src_v)
            pltpu.sync_copy(dst_h.at[pl.ds(base, EC0)], dst_v)
            pltpu.sync_copy(w_h.at[pl.ds(base, EC0)], w_v)

            def grp(i, c2):
                s16 = src_v[pl.ds(i * 16, 16)]
                d16 = dst_v[pl.ds(i * 16, 16)]
                xs = plsc.load_gather(x_v, [s16])
                xd = plsc.load_gather(x_v, [d16])
                ys = plsc.load_gather(y_v, [s16])
                yd = plsc.load_gather(y_v, [d16])
                dx = xs - xd
                dy = ys - yd
                r2_v[pl.ds(i * 16, 16)] = dx * dx + dy * dy
                return c2

            lax.fori_loop(0, EC0 // 16, grp, 0)
            pltpu.sync_copy(r2_v, r2_h.at[pl.ds(base, EC0)])
            pltpu.sync_copy(w_v, ws_sh.at[dst_v], add=True)
            return carry

        lax.fori_loop(0, NCHUNK0, chunk, 0)
        plsc.subcore_barrier()
        pltpu.sync_copy(ws_sh.at[pl.ds(tid * STRIP, STRIP)],
                        ws_h.at[cid, pl.ds(tid * STRIP, STRIP)])

    return k(x, y, src, dst, w)


def _sc_layer(a, b, c, sdw):
    @functools.partial(
        pl.kernel,
        out_type=jax.ShapeDtypeStruct((NC, NPAD, D_H), jnp.float32),
        mesh=_sc_mesh(),
        scratch_types=(
            [pltpu.VMEM((4, 3, EC), jnp.int32),
             pltpu.VMEM((4, EC, D_H), jnp.float32),
             pltpu.VMEM_SHARED((NPAD, D_H), jnp.float32)]
            + [pltpu.SemaphoreType.DMA] * 16
        ),
        compiler_params=pltpu.CompilerParams(needs_layout_passes=False),
    )
    def k(a_h, b_h, c_h, sdw_h, s_out,
          idx_v, m_v, s_sh, *sems):
        cid = lax.axis_index("c")
        tid = lax.axis_index("s")
        wid = cid * NS + tid
        sem_ic = sems[0:4]
        sem_a = sems[4:8]
        sem_b = sems[8:12]
        sem_s = sems[12:16]

        zero16 = jnp.zeros((16,), jnp.float32)

        def zz(i, carry):
            for j in range(D_H // 16):
                m_v[0, i, pl.ds(j * 16, 16)] = zero16
            return carry

        lax.fori_loop(0, EC, zz, 0)
        for kk in range(STRIP // EC):
            pltpu.sync_copy(m_v.at[0],
                            s_sh.at[pl.ds(tid * STRIP + kk * EC, EC)])
        plsc.subcore_barrier()

        def issue_idx_c(g, p):
            base = wid * E_PER_W + g * EC
            pltpu.async_copy(sdw_h.at[wid * NCHUNK + g], idx_v.at[p],
                             sem_ic[p])
            pltpu.async_copy(c_h.at[pl.ds(base, EC)], m_v.at[p], sem_ic[p])

        def wait_idx_c(p):
            pltpu.make_async_copy(sdw_h.at[0], idx_v.at[p], sem_ic[p]).wait()
            pltpu.make_async_copy(c_h.at[pl.ds(0, EC)], m_v.at[p],
                                  sem_ic[p]).wait()

        def issue_a(p):
            pltpu.async_copy(a_h.at[idx_v.at[p, 0]], m_v.at[p], sem_a[p],
                             add=True)

        def wait_a(p):
            pltpu.make_async_copy(a_h.at[idx_v.at[p, 0]], m_v.at[p],
                                  sem_a[p]).wait()

        def issue_b(p):
            pltpu.async_copy(b_h.at[idx_v.at[p, 1]], m_v.at[p], sem_b[p],
                             add=True)

        def wait_b(p):
            pltpu.make_async_copy(b_h.at[idx_v.at[p, 1]], m_v.at[p],
                                  sem_b[p]).wait()

        def compute(p):
            def grp(g16, carry):
                pos = (jnp.full((16,), g16 * 16, jnp.int32)
                       + lax.iota(jnp.int32, 16))
                w16 = plsc.bitcast(
                    plsc.load_gather(idx_v.at[p, 2], [pos]), jnp.float32)
                for l in range(16):
                    e = g16 * 16 + l
                    wv = jnp.take_along_axis(
                        w16, jnp.full((16,), l, jnp.int32), axis=0)
                    for j in range(D_H // 16):
                        sl = pl.ds(j * 16, 16)
                        m_v[p, e, sl] = jnp.maximum(m_v[p, e, sl] * wv, 0.0)
                return carry

            lax.fori_loop(0, EC // 16, grp, 0)

        def issue_scatter(p):
            pltpu.async_copy(m_v.at[p], s_sh.at[idx_v.at[p, 1]], sem_s[p],
                             add=True)

        def wait_scatter(p):
            pltpu.make_async_copy(m_v.at[p], s_sh.at[idx_v.at[p, 1]],
                                  sem_s[p]).wait()

        def chunk(g, p, first=False, has1=True, has2=True, has3=True):
            wait_b(p)
            compute(p)
            issue_scatter(p)
            if has1:
                wait_a((p + 1) % 4)
                issue_b((p + 1) % 4)
            if has2:
                wait_idx_c((p + 2) % 4)
                issue_a((p + 2) % 4)
            if not first:
                wait_scatter((p + 3) % 4)
                if has3:
                    issue_idx_c(g + 3, (p + 3) % 4)
            elif has3:
                issue_idx_c(g + 3, (p + 3) % 4)

        issue_idx_c(0, 0)
        issue_idx_c(1, 1)
        issue_idx_c(2, 2)
        wait_idx_c(0)
        issue_a(0)
        wait_a(0)
        issue_b(0)
        wait_idx_c(1)
        issue_a(1)

        chunk(0, 0, first=True)

        def blk(i, carry):
            g = 1 + i * 4
            chunk(g, 1)
            chunk(g + 1, 2)
            chunk(g + 2, 3)
            chunk(g + 3, 0)
            return carry

        lax.fori_loop(0, (NCHUNK - 6) // 4, blk, 0)
        for g in range(NCHUNK - 5, NCHUNK):
            chunk(g, g % 4, has1=g + 1 < NCHUNK, has2=g + 2 < NCHUNK,
                  has3=g + 3 < NCHUNK)
        wait_scatter((NCHUNK - 1) % 4)
        plsc.subcore_barrier()
        for kk in range(STRIP // 128):
            base = tid * STRIP + kk * 128
            pltpu.sync_copy(s_sh.at[pl.ds(base, 128)],
                            s_out.at[cid, pl.ds(base, 128)])

    return k(a, b, c, sdw)


def _tc_rbf(r2, w1r, b1):
    EB = 2000

    def body(r2_ref, wr_ref, br_ref, c_ref):
        r = jnp.sqrt(r2_ref[...] + 1e-8)
        centers = (lax.broadcasted_iota(jnp.int32, (1, N_RBF), 1)
                   .astype(jnp.float32) * np.float32(_SQRT2 / (N_RBF - 1)))
        d = r - centers
        rbf = jnp.exp(d * d * _NEG_INV_2W2)
        c_ref[...] = jnp.dot(rbf, wr_ref[...],
                             preferred_element_type=jnp.float32) + br_ref[...]

    return pl.pallas_call(
        body,
        grid=(N_EDGES // EB,),
        in_specs=[
            pl.BlockSpec((EB, 1), lambda i: (i, 0)),
            pl.BlockSpec((N_RBF, D_H), lambda i: (0, 0)),
            pl.BlockSpec((1, D_H), lambda i: (0, 0)),
        ],
        out_specs=pl.BlockSpec((EB, D_H), lambda i: (i, 0)),
        out_shape=jax.ShapeDtypeStruct((N_EDGES, D_H), jnp.float32),
    )(r2, w1r, b1)


def _tc_node_in(xyr01, ws, w0, b0, w1, b1, wab):
    def body(xyr_ref, ws_ref, w0_ref, b0_ref, w1_ref, b1_ref, wab_ref,
             h_ref, a_ref, b_ref):
        ws_v = ws_ref[...]
        deg = ws_v / jnp.maximum(jnp.mean(ws_v), 1.0)
        x0 = jnp.concatenate([xyr_ref[...], deg], axis=1)
        t = jnp.maximum(
            jnp.dot(x0, w0_ref[...], preferred_element_type=jnp.float32)
            + b0_ref[...], 0.0)
        h = jnp.dot(t, w1_ref[...],
                    preferred_element_type=jnp.float32) + b1_ref[...]
        h_ref[...] = h
        ab = jnp.dot(h, wab_ref[...], preferred_element_type=jnp.float32)
        a_ref[...] = ab[:, :D_H]
        b_ref[...] = ab[:, D_H:]

    return pl.pallas_call(
        body,
        out_shape=[jax.ShapeDtypeStruct((N_NODES, D_H), jnp.float32)] * 3,
    )(xyr01, ws, w0, b0, w1, b1, wab)


def _tc_post(h, s2, ws, w2, b2, v1, c1, v2, c2, lng, lnb, wab_next):
    has_next = wab_next is not None

    def body(*refs):
        (h_ref, s_ref, ws_ref, w2_ref, b2_ref, v1_ref, c1_ref, v2_ref,
         c2_ref, g_ref, bb_ref) = refs[:11]
        if has_next:
            wab_ref, hn_ref, a_ref, b_ref = refs[11:]
        else:
            hn_ref = refs[11]
        h_v = h_ref[...]
        s = s_ref[0, :N_NODES, :] + s_ref[1, :N_NODES, :]
        agg = (jnp.dot(s, w2_ref[...], preferred_element_type=jnp.float32)
               + ws_ref[...] * b2_ref[...])
        cat = jnp.concatenate([h_v, agg], axis=1)
        t = jnp.maximum(
            jnp.dot(cat, v1_ref[...], preferred_element_type=jnp.float32)
            + c1_ref[...], 0.0)
        u = jnp.dot(t, v2_ref[...],
                    preferred_element_type=jnp.float32) + c2_ref[...]
        y = h_v + u
        mu = jnp.mean(y, axis=1, keepdims=True)
        var = jnp.mean((y - mu) ** 2, axis=1, keepdims=True)
        hn = (y - mu) / jnp.sqrt(var + 1e-5) * g_ref[...] + bb_ref[...]
        hn_ref[...] = hn
        if has_next:
            ab = jnp.dot(hn, wab_ref[...], preferred_element_type=jnp.float32)
            a_ref[...] = ab[:, :D_H]
            b_ref[...] = ab[:, D_H:]

    n_out = 3 if has_next else 1
    args = [h, s2, ws, w2, b2, v1, c1, v2, c2, lng, lnb]
    if has_next:
        args.append(wab_next)
    return pl.pallas_call(
        body,
        out_shape=[jax.ShapeDtypeStruct((N_NODES, D_H), jnp.float32)] * n_out,
    )(*args)


def _tc_head(h, xyr01, ws, wh0, bh0, wh1, bh1):
    def body(h_ref, xyr_ref, ws_ref, w0_ref, b0_ref, w1_ref, b1_ref, o_ref):
        h_v = h_ref[...]
        pm = jnp.mean(h_v, axis=0, keepdims=True)
        px = jnp.max(h_v, axis=0, keepdims=True)
        xyr = xyr_ref[...]
        one = jnp.ones((1, 1), jnp.float32)
        em = jnp.sum(ws_ref[...]) * (0.5 / float(N_NODES))
        col2 = xyr[:, 2:3]
        mu2 = jnp.mean(col2)
        sd2 = jnp.sqrt(jnp.mean((col2 - mu2) ** 2))
        xy = xyr[:, :2]
        muxy = jnp.mean(xy)
        sdxy = jnp.sqrt(jnp.mean((xy - muxy) ** 2))
        stats = jnp.concatenate(
            [one * _LOG_N, one * em, one * mu2, one * sd2, one * sdxy], axis=1)
        z = jnp.concatenate([pm, px, stats], axis=1)
        t = jnp.maximum(
            jnp.dot(z, w0_ref[...], preferred_element_type=jnp.float32)
            + b0_ref[...], 0.0)
        o_ref[...] = jnp.dot(t, w1_ref[...],
                             preferred_element_type=jnp.float32) + b1_ref[...]

    return pl.pallas_call(
        body,
        out_shape=jax.ShapeDtypeStruct((1, 1), jnp.float32),
    )(h, xyr01, ws, wh0, bh0, wh1, bh1)


def kernel(xyr01, edge_index, edge_weight, params):
    x = xyr01[:, 0]
    y = xyr01[:, 1]
    src = edge_index[0]
    dst = edge_index[1]
    w = edge_weight

    node_in = params['node_in']
    w0 = node_in[0]['W']
    b0 = node_in[0]['b'].reshape(1, D_H)
    w1 = node_in[1]['W']
    b1n = node_in[1]['b'].reshape(1, D_H)

    layers = params['layers']
    wab, w1r, b1m, w2, b2, v1, c1, v2, c2, lng, lnb = ([] for _ in range(11))
    for lp in layers:
        W1 = lp['phi_m'][0]['W']
        wab.append(jnp.concatenate([W1[:D_H], W1[D_H:2 * D_H]], axis=1))
        w1r.append(W1[2 * D_H:])
        b1m.append(lp['phi_m'][0]['b'])
        w2.append(lp['phi_m'][1]['W'])
        b2.append(lp['phi_m'][1]['b'].reshape(1, D_H))
        v1.append(lp['phi_h'][0]['W'])
        c1.append(lp['phi_h'][0]['b'].reshape(1, D_H))
        v2.append(lp['phi_h'][1]['W'])
        c2.append(lp['phi_h'][1]['b'].reshape(1, D_H))
        lng.append(lp['ln_scale'].reshape(1, D_H))
        lnb.append(lp['ln_bias'].reshape(1, D_H))

    head = params['head']
    wh0 = head[0]['W']
    bh0 = head[0]['b'].reshape(1, D_H)
    wh1 = head[1]['W']
    bh1 = head[1]['b'].reshape(1, 1)

    sdw = jnp.stack(
        [src.reshape(-1, EC), dst.reshape(-1, EC),
         lax.bitcast_convert_type(w, jnp.int32).reshape(-1, EC)], axis=1)

    r2, ws2 = _sc_pass0(x, y, src, dst, w)
    ws = (ws2[0] + ws2[1])[:N_NODES].reshape(N_NODES, 1)

    r2c = r2.reshape(N_EDGES, 1)
    c_all = [_tc_rbf(r2c, w1r[l], b1m[l].reshape(1, D_H))
             for l in range(len(layers))]

    h, a_n, b_n = _tc_node_in(xyr01, ws, w0, b0, w1, b1n, wab[0])

    for l in range(len(layers)):
        s2 = _sc_layer(a_n, b_n, c_all[l], sdw)
        nxt = wab[l + 1] if l + 1 < len(layers) else None
        outs = _tc_post(h, s2, ws, w2[l], b2[l], v1[l], c1[l], v2[l], c2[l],
                        lng[l], lnb[l], nxt)
        if nxt is not None:
            h, a_n, b_n = outs
        else:
            h, = outs

    out = _tc_head(h, xyr01, ws, wh0, bh0, wh1, bh1)
    return out.reshape(())
